# baseline (device time: 254618 ns/iter reference)
import jax
import jax.numpy as jnp
from jax import lax
from jax.experimental import pallas as pl
from jax.experimental.pallas import tpu as pltpu

N_DEV = 32
M_PER = 128
K = 4096
N_PER = 64


def kernel(x, w_mat, scale_x, scale_w):
    x8 = x.astype(jnp.float8_e4m3fn)
    w16 = w_mat.astype(jnp.bfloat16)
    scale = (scale_x[0] * scale_w[0]).reshape(1, 1)

    def body(x_ref, w_ref, scale_ref, out_ref, comm_ref, send_sems, recv_sems):
        my = lax.axis_index("i")
        left = lax.rem(my + (N_DEV - 1), N_DEV)
        right = lax.rem(my + 1, N_DEV)

        barrier = pltpu.get_barrier_semaphore()
        for nbr in (left, right):
            pl.semaphore_signal(
                barrier, inc=1,
                device_id=(nbr,), device_id_type=pl.DeviceIdType.MESH,
            )
        pl.semaphore_wait(barrier, 2)

        comm_ref[pl.ds(my * M_PER, M_PER), :] = x_ref[...]

        for h in range(N_DEV - 1):
            o = lax.rem(my - h + N_DEV, N_DEV)
            rdma = pltpu.make_async_remote_copy(
                src_ref=comm_ref.at[pl.ds(o * M_PER, M_PER), :],
                dst_ref=comm_ref.at[pl.ds(o * M_PER, M_PER), :],
                send_sem=send_sems.at[h],
                recv_sem=recv_sems.at[h],
                device_id=(right,),
                device_id_type=pl.DeviceIdType.MESH,
            )
            rdma.start()
            rdma.wait()

        s = scale_ref[0, 0]
        for c in range(N_DEV):
            a = comm_ref[c * M_PER:(c + 1) * M_PER, :].astype(jnp.bfloat16)
            acc = jnp.dot(a, w_ref[...], preferred_element_type=jnp.float32)
            out_ref[c * M_PER:(c + 1) * M_PER, :] = acc * s

    return pl.pallas_call(
        body,
        out_shape=jax.ShapeDtypeStruct((N_DEV * M_PER, N_PER), jnp.float32),
        in_specs=[
            pl.BlockSpec(memory_space=pltpu.VMEM),
            pl.BlockSpec(memory_space=pltpu.VMEM),
            pl.BlockSpec(memory_space=pltpu.SMEM),
        ],
        out_specs=pl.BlockSpec(memory_space=pltpu.VMEM),
        scratch_shapes=[
            pltpu.VMEM((N_DEV * M_PER, K), jnp.float8_e4m3fn),
            pltpu.SemaphoreType.DMA((N_DEV - 1,)),
            pltpu.SemaphoreType.DMA((N_DEV - 1,)),
        ],
        compiler_params=pltpu.CompilerParams(
            collective_id=0,
            vmem_limit_bytes=100 * 1024 * 1024,
        ),
    )(x8, w16, scale)


# device time: 194431 ns/iter; 1.3096x vs baseline; 1.3096x over previous
import jax
import jax.numpy as jnp
from jax import lax
from jax.experimental import pallas as pl
from jax.experimental.pallas import tpu as pltpu

N_DEV = 32
M_PER = 128
SUB = M_PER // 2
K = 4096
N_PER = 64

R_HOPS = 16
L_HOPS = 15


def kernel(x, w_mat, scale_x, scale_w):
    x8 = x.astype(jnp.float8_e4m3fn)
    w16 = w_mat.astype(jnp.bfloat16)
    scale = (scale_x[0] * scale_w[0]).reshape(1, 1)

    def body(x_ref, w_ref, scale_ref, out_ref, comm_ref,
             rs_send, rs_recv, ls_send, ls_recv):
        my = lax.axis_index("i")
        left = lax.rem(my + (N_DEV - 1), N_DEV)
        right = lax.rem(my + 1, N_DEV)

        barrier = pltpu.get_barrier_semaphore()
        for nbr in (left, right):
            pl.semaphore_signal(
                barrier, inc=1,
                device_id=(nbr,), device_id_type=pl.DeviceIdType.MESH,
            )
        pl.semaphore_wait(barrier, 2)

        comm_ref[pl.ds(my * M_PER, M_PER), :] = x_ref[...]

        def send(origin, sub, dev, send_sem, recv_sem):
            sl = pl.ds(origin * M_PER + sub * SUB, SUB)
            rdma = pltpu.make_async_remote_copy(
                src_ref=comm_ref.at[sl, :],
                dst_ref=comm_ref.at[sl, :],
                send_sem=send_sem,
                recv_sem=recv_sem,
                device_id=(dev,),
                device_id_type=pl.DeviceIdType.MESH,
            )
            rdma.start()
            return rdma

        s = scale_ref[0, 0]

        def gemm(origin):
            sl = pl.ds(origin * M_PER, M_PER)
            a = comm_ref[sl, :].astype(jnp.bfloat16)
            acc = jnp.dot(a, w_ref[...], preferred_element_type=jnp.float32)
            out_ref[sl, :] = acc * s

        sends = []
        for sub in range(2):
            sends.append(send(my, sub, right, rs_send.at[0, sub],
                              rs_recv.at[0, sub]))
            sends.append(send(my, sub, left, ls_send.at[0, sub],
                              ls_recv.at[0, sub]))
        gemm(my)

        recv_only = []
        for h in range(1, R_HOPS + 1):
            o_r = lax.rem(my - h + N_DEV, N_DEV)
            for sub in range(2):
                rcv = pltpu.make_async_remote_copy(
                    src_ref=comm_ref.at[pl.ds(o_r * M_PER + sub * SUB, SUB), :],
                    dst_ref=comm_ref.at[pl.ds(o_r * M_PER + sub * SUB, SUB), :],
                    send_sem=rs_send.at[h - 1, sub],
                    recv_sem=rs_recv.at[h - 1, sub],
                    device_id=(left,),
                    device_id_type=pl.DeviceIdType.MESH,
                )
                rcv.wait_recv()
                if h < R_HOPS:
                    sends.append(send(o_r, sub, right, rs_send.at[h, sub],
                                      rs_recv.at[h, sub]))
            if h <= L_HOPS:
                o_l = lax.rem(my + h, N_DEV)
                for sub in range(2):
                    rcv = pltpu.make_async_remote_copy(
                        src_ref=comm_ref.at[pl.ds(o_l * M_PER + sub * SUB, SUB), :],
                        dst_ref=comm_ref.at[pl.ds(o_l * M_PER + sub * SUB, SUB), :],
                        send_sem=ls_send.at[h - 1, sub],
                        recv_sem=ls_recv.at[h - 1, sub],
                        device_id=(right,),
                        device_id_type=pl.DeviceIdType.MESH,
                    )
                    rcv.wait_recv()
                    if h < L_HOPS:
                        sends.append(send(o_l, sub, left, ls_send.at[h, sub],
                                          ls_recv.at[h, sub]))
                gemm(o_l)
            gemm(o_r)

        for rdma in sends:
            rdma.wait_send()

    return pl.pallas_call(
        body,
        out_shape=jax.ShapeDtypeStruct((N_DEV * M_PER, N_PER), jnp.float32),
        in_specs=[
            pl.BlockSpec(memory_space=pltpu.VMEM),
            pl.BlockSpec(memory_space=pltpu.VMEM),
            pl.BlockSpec(memory_space=pltpu.SMEM),
        ],
        out_specs=pl.BlockSpec(memory_space=pltpu.VMEM),
        scratch_shapes=[
            pltpu.VMEM((N_DEV * M_PER, K), jnp.float8_e4m3fn),
            pltpu.SemaphoreType.DMA((R_HOPS, 2)),
            pltpu.SemaphoreType.DMA((R_HOPS, 2)),
            pltpu.SemaphoreType.DMA((L_HOPS, 2)),
            pltpu.SemaphoreType.DMA((L_HOPS, 2)),
        ],
        compiler_params=pltpu.CompilerParams(
            collective_id=0,
            vmem_limit_bytes=100 * 1024 * 1024,
        ),
    )(x8, w16, scale)


# device time: 106088 ns/iter; 2.4001x vs baseline; 1.8327x over previous
import jax
import jax.numpy as jnp
from jax import lax
from jax.experimental import pallas as pl
from jax.experimental.pallas import tpu as pltpu

N_DEV = 32
M_PER = 128
SUB = M_PER // 2
K = 4096
N_PER = 64

R_HOPS = 16
L_HOPS = 15


def _hamiltonian_ring():
    cyc = []
    for z in range(4):
        ys = range(4) if z % 2 == 0 else range(3, -1, -1)
        cyc += [(1, y, z) for y in ys]
    for z in range(3, -1, -1):
        ys = range(4) if z % 2 == 1 else range(3, -1, -1)
        cyc += [(0, y, z) for y in ys]

    def lidx(x, y, z):
        return 8 * z + 2 * y + (x if y % 2 == 0 else 1 - x)

    ring = [lidx(*c) for c in cyc]
    assert sorted(ring) == list(range(N_DEV))
    nxt = [0] * N_DEV
    prv = [0] * N_DEV
    for j, m in enumerate(ring):
        nxt[m] = ring[(j + 1) % N_DEV]
        prv[m] = ring[(j - 1) % N_DEV]
    return nxt, prv


def kernel(x, w_mat, scale_x, scale_w):
    x8 = x.astype(jnp.float8_e4m3fn)
    w16 = w_mat.astype(jnp.bfloat16)
    scale = (scale_x[0] * scale_w[0]).reshape(1, 1)
    nxt_l, prv_l = _hamiltonian_ring()
    nxt_tbl = jnp.asarray(nxt_l, dtype=jnp.int32)
    prv_tbl = jnp.asarray(prv_l, dtype=jnp.int32)

    def body(x_ref, w_ref, scale_ref, nxt_ref, prv_ref, out_ref, comm_ref,
             rs_send, rs_recv, ls_send, ls_recv):
        my = lax.axis_index("i")
        left = prv_ref[my]
        right = nxt_ref[my]

        barrier = pltpu.get_barrier_semaphore()
        for nbr in (left, right):
            pl.semaphore_signal(
                barrier, inc=1,
                device_id=(nbr,), device_id_type=pl.DeviceIdType.MESH,
            )
        pl.semaphore_wait(barrier, 2)

        comm_ref[pl.ds(my * M_PER, M_PER), :] = x_ref[...]

        def send(origin, sub, dev, send_sem, recv_sem):
            sl = pl.ds(origin * M_PER + sub * SUB, SUB)
            rdma = pltpu.make_async_remote_copy(
                src_ref=comm_ref.at[sl, :],
                dst_ref=comm_ref.at[sl, :],
                send_sem=send_sem,
                recv_sem=recv_sem,
                device_id=(dev,),
                device_id_type=pl.DeviceIdType.MESH,
            )
            rdma.start()
            return rdma

        s = scale_ref[0, 0]

        def gemm(origin):
            sl = pl.ds(origin * M_PER, M_PER)
            a = comm_ref[sl, :].astype(jnp.bfloat16)
            acc = jnp.dot(a, w_ref[...], preferred_element_type=jnp.float32)
            out_ref[sl, :] = acc * s

        sends = []
        for sub in range(2):
            sends.append(send(my, sub, right, rs_send.at[0, sub],
                              rs_recv.at[0, sub]))
            sends.append(send(my, sub, left, ls_send.at[0, sub],
                              ls_recv.at[0, sub]))
        gemm(my)

        o_r = my
        o_l = my
        for h in range(1, R_HOPS + 1):
            o_r = prv_ref[o_r]
            for sub in range(2):
                rcv = pltpu.make_async_remote_copy(
                    src_ref=comm_ref.at[pl.ds(o_r * M_PER + sub * SUB, SUB), :],
                    dst_ref=comm_ref.at[pl.ds(o_r * M_PER + sub * SUB, SUB), :],
                    send_sem=rs_send.at[h - 1, sub],
                    recv_sem=rs_recv.at[h - 1, sub],
                    device_id=(left,),
                    device_id_type=pl.DeviceIdType.MESH,
                )
                rcv.wait_recv()
                if h < R_HOPS:
                    sends.append(send(o_r, sub, right, rs_send.at[h, sub],
                                      rs_recv.at[h, sub]))
            if h <= L_HOPS:
                o_l = nxt_ref[o_l]
                for sub in range(2):
                    rcv = pltpu.make_async_remote_copy(
                        src_ref=comm_ref.at[pl.ds(o_l * M_PER + sub * SUB, SUB), :],
                        dst_ref=comm_ref.at[pl.ds(o_l * M_PER + sub * SUB, SUB), :],
                        send_sem=ls_send.at[h - 1, sub],
                        recv_sem=ls_recv.at[h - 1, sub],
                        device_id=(right,),
                        device_id_type=pl.DeviceIdType.MESH,
                    )
                    rcv.wait_recv()
                    if h < L_HOPS:
                        sends.append(send(o_l, sub, left, ls_send.at[h, sub],
                                          ls_recv.at[h, sub]))
                gemm(o_l)
            gemm(o_r)

        for rdma in sends:
            rdma.wait_send()

    return pl.pallas_call(
        body,
        out_shape=jax.ShapeDtypeStruct((N_DEV * M_PER, N_PER), jnp.float32),
        in_specs=[
            pl.BlockSpec(memory_space=pltpu.VMEM),
            pl.BlockSpec(memory_space=pltpu.VMEM),
            pl.BlockSpec(memory_space=pltpu.SMEM),
            pl.BlockSpec(memory_space=pltpu.SMEM),
            pl.BlockSpec(memory_space=pltpu.SMEM),
        ],
        out_specs=pl.BlockSpec(memory_space=pltpu.VMEM),
        scratch_shapes=[
            pltpu.VMEM((N_DEV * M_PER, K), jnp.float8_e4m3fn),
            pltpu.SemaphoreType.DMA((R_HOPS, 2)),
            pltpu.SemaphoreType.DMA((R_HOPS, 2)),
            pltpu.SemaphoreType.DMA((L_HOPS, 2)),
            pltpu.SemaphoreType.DMA((L_HOPS, 2)),
        ],
        compiler_params=pltpu.CompilerParams(
            collective_id=0,
            vmem_limit_bytes=100 * 1024 * 1024,
        ),
    )(x8, w16, scale, nxt_tbl, prv_tbl)
